# baseline (speedup 1.0000x reference)
import functools

import jax
import jax.numpy as jnp
from jax import lax
from jax.experimental import pallas as pl
from jax.experimental.pallas import tpu as pltpu
from jax.experimental.pallas import tpu_sc as plsc

NC = 2
NS = 16
L = 16
NW = NC * NS

B = 16384
D = 32
BPW = B // NW
CHUNK = 128
NCH = BPW // CHUNK

_mesh = plsc.VectorSubcoreMesh(
    core_axis_name="c", subcore_axis_name="s", num_cores=NC, num_subcores=NS
)


@functools.partial(
    pl.kernel,
    out_type=jax.ShapeDtypeStruct((B,), jnp.float32),
    mesh=_mesh,
    compiler_params=pltpu.CompilerParams(
        needs_layout_passes=False, use_tc_tiling_on_sc=False),
    scratch_types=[
        pltpu.VMEM((NCH, CHUNK), jnp.int32),
        pltpu.VMEM((NCH, CHUNK), jnp.int32),
        pltpu.VMEM((BPW, D), jnp.float32),
        pltpu.VMEM((BPW, D), jnp.float32),
        pltpu.VMEM((BPW,), jnp.float32),
        pltpu.SemaphoreType.DMA,
    ],
)
def _cf_ratings(uid_hbm, iid_hbm, ut_hbm, it_hbm, out_hbm,
                uidx, iidx, urows, irows, outv, sem):
    wid = lax.axis_index("s") * NC + lax.axis_index("c")

    pltpu.sync_copy(uid_hbm.at[pl.ds(wid * NCH, NCH)], uidx)
    pltpu.sync_copy(iid_hbm.at[pl.ds(wid * NCH, NCH)], iidx)

    copies = []
    for j in range(NCH):
        copies.append(pltpu.async_copy(
            ut_hbm.at[uidx.at[j]], urows.at[pl.ds(j * CHUNK, CHUNK)], sem))
        copies.append(pltpu.async_copy(
            it_hbm.at[iidx.at[j]], irows.at[pl.ds(j * CHUNK, CHUNK)], sem))
    for c in copies:
        c.wait()

    lanes = lax.iota(jnp.int32, L)

    def group_body(g, carry):
        row0 = g * L
        rows = row0 + lanes
        accs = [jnp.zeros((L,), jnp.float32) for _ in range(4)]
        for d in range(D):
            col = jnp.full((L,), d, jnp.int32)
            u = plsc.load_gather(urows, [rows, col])
            v = plsc.load_gather(irows, [rows, col])
            accs[d % 4] = accs[d % 4] + u * v
        outv[pl.ds(row0, L)] = (accs[0] + accs[1]) + (accs[2] + accs[3])
        return carry

    lax.fori_loop(0, BPW // L, group_body, 0)

    pltpu.sync_copy(outv, out_hbm.at[pl.ds(wid * BPW, BPW)])


def kernel(user_ids, item_ids, user_table, item_table):
    uid = user_ids.astype(jnp.int32).reshape(NW * NCH, CHUNK)
    iid = item_ids.astype(jnp.int32).reshape(NW * NCH, CHUNK)
    return _cf_ratings(uid, iid, user_table, item_table)


# speedup vs baseline: 3.2267x; 3.2267x over previous
import functools

import jax
import jax.numpy as jnp
from jax import lax
from jax.experimental import pallas as pl
from jax.experimental.pallas import tpu as pltpu
from jax.experimental.pallas import tpu_sc as plsc

NC = 2
NS = 16
L = 16
NW = NC * NS

B = 16384
D = 32
V = 1_000_000
VMAIN = 999_936
VTAIL = V - VMAIN
BPW = B // NW
CHUNK = 128
NCH = BPW // CHUNK

TPB = 93
WS = TPB * 128
NTILES = VMAIN // 128
NCOLS = NTILES // TPB
REGION = NTILES * 1024
MSIZE = 4 * REGION
NSLABS = 2 * 4 * NCOLS
SLABS_PER_W = NSLABS // NW
QMAX = NTILES - 1
GSPAN = QMAX * 1024 + 1024

_mesh = plsc.VectorSubcoreMesh(
    core_axis_name="c", subcore_axis_name="s", num_cores=NC, num_subcores=NS
)


@functools.partial(
    pl.kernel,
    out_type=(
        jax.ShapeDtypeStruct((MSIZE // 1024, 8, 128), jnp.float32),
        jax.ShapeDtypeStruct((MSIZE // 1024, 8, 128), jnp.float32),
    ),
    mesh=_mesh,
    compiler_params=pltpu.CompilerParams(
        needs_layout_passes=False, use_tc_tiling_on_sc=True),
    scratch_types=[
        pltpu.VMEM((TPB, 8, 128), jnp.float32),
        pltpu.SemaphoreType.DMA,
    ],
)
def _repack(ut_hbm, it_hbm, uout, iout, buf, sem):
    wid = lax.axis_index("s") * NC + lax.axis_index("c")

    def slab_body(i, carry):
        k = wid * SLABS_PER_W + i
        t = k // (4 * NCOLS)
        rem = k % (4 * NCOLS)
        r = rem // NCOLS
        c = rem % NCOLS
        row8 = r * 8
        tile0 = r * NTILES + c * TPB

        @pl.when(t == 0)
        def _():
            cps = [pltpu.async_copy(
                ut_hbm.at[pl.ds(row8, 8), pl.ds((c * TPB + j) * 128, 128)],
                buf.at[j], sem) for j in range(TPB)]
            for cp in cps:
                cp.wait()
            pltpu.sync_copy(buf, uout.at[pl.ds(tile0, TPB)])

        @pl.when(t == 1)
        def _():
            cps = [pltpu.async_copy(
                it_hbm.at[pl.ds(row8, 8), pl.ds((c * TPB + j) * 128, 128)],
                buf.at[j], sem) for j in range(TPB)]
            for cp in cps:
                cp.wait()
            pltpu.sync_copy(buf, iout.at[pl.ds(tile0, TPB)])

        return carry

    lax.fori_loop(0, SLABS_PER_W, slab_body, 0)


@functools.partial(
    pl.kernel,
    out_type=jax.ShapeDtypeStruct((B,), jnp.float32),
    mesh=_mesh,
    compiler_params=pltpu.CompilerParams(
        needs_layout_passes=False, use_tc_tiling_on_sc=False),
    scratch_types=[
        pltpu.VMEM((BPW,), jnp.int32),
        pltpu.VMEM((BPW,), jnp.int32),
        pltpu.VMEM((BPW,), jnp.int32),
        pltpu.VMEM((BPW,), jnp.int32),
        pltpu.VMEM((D, BPW), jnp.float32),
        pltpu.VMEM((D, BPW), jnp.float32),
        pltpu.VMEM((VTAIL * D,), jnp.float32),
        pltpu.VMEM((VTAIL * D,), jnp.float32),
        pltpu.VMEM((BPW,), jnp.float32),
        pltpu.SemaphoreType.DMA,
    ],
)
def _cf_ratings(uid_hbm, iid_hbm, ut_hbm, it_hbm, tu_hbm, ti_hbm, out_hbm,
                uidx, iidx, upart, ipart, ucols, icols, utail, itail,
                outv, sem):
    wid = lax.axis_index("s") * NC + lax.axis_index("c")
    base = wid * BPW

    pltpu.sync_copy(uid_hbm.at[pl.ds(base, BPW)], uidx)
    pltpu.sync_copy(iid_hbm.at[pl.ds(base, BPW)], iidx)
    pltpu.sync_copy(tu_hbm, utail)
    pltpu.sync_copy(ti_hbm, itail)

    def part_body(g, carry):
        sl = pl.ds(g * L, L)
        for idv, pv in ((uidx, upart), (iidx, ipart)):
            ids = idv[sl]
            q = jnp.minimum(lax.shift_right_logical(ids, 7), QMAX)
            pv[sl] = q * 1024 + lax.bitwise_and(ids, 127)
        outv[pl.ds(g * L, L)] = jnp.zeros((L,), jnp.float32)
        return carry
    lax.fori_loop(0, BPW // L, part_body, 0)

    def d_loop(d, carry):
        offs = (d // 8) * REGION + (d % 8) * 128
        cps = []
        for c in range(NCH):
            sl = pl.ds(c * CHUNK, CHUNK)
            cps.append(pltpu.async_copy(
                ut_hbm.at[pl.ds(offs, GSPAN)].at[upart.at[sl]],
                ucols.at[d, sl], sem))
            cps.append(pltpu.async_copy(
                it_hbm.at[pl.ds(offs, GSPAN)].at[ipart.at[sl]],
                icols.at[d, sl], sem))
        for cp in cps:
            cp.wait()

        def group_body(g, c2):
            sl = pl.ds(g * L, L)
            outv[sl] = outv[sl] + ucols[d, sl] * icols[d, sl]
            return c2
        lax.fori_loop(0, BPW // L, group_body, 0)
        return carry

    lax.fori_loop(0, D, d_loop, 0)

    def fix_body(g, carry):
        sl = pl.ds(g * L, L)
        uids = uidx[sl]
        iids = iidx[sl]
        um = uids >= VMAIN
        im = iids >= VMAIN
        any_tail = lax.reduce_max(
            jnp.where(um | im, jnp.int32(1), jnp.int32(0)), axes=(0,))

        @pl.when(any_tail > 0)
        def _():
            ubase = jnp.where(um, uids - VMAIN, 0) * D
            ibase = jnp.where(im, iids - VMAIN, 0) * D
            acc = jnp.zeros((L,), jnp.float32)
            for d in range(D):
                tu = plsc.load_gather(utail, [ubase + d])
                ti = plsc.load_gather(itail, [ibase + d])
                u = jnp.where(um, tu, ucols[d, sl])
                v = jnp.where(im, ti, icols[d, sl])
                acc = acc + u * v
            outv[sl] = acc
        return carry
    lax.fori_loop(0, BPW // L, fix_body, 0)

    pltpu.sync_copy(outv, out_hbm.at[pl.ds(base, BPW)])


def kernel(user_ids, item_ids, user_table, item_table):
    uid = user_ids.astype(jnp.int32)
    iid = item_ids.astype(jnp.int32)
    utm, itm = _repack(user_table.T, item_table.T)
    utm = utm.reshape(MSIZE)
    itm = itm.reshape(MSIZE)
    tail_u = user_table[VMAIN:].reshape(VTAIL * D)
    tail_i = item_table[VMAIN:].reshape(VTAIL * D)
    return _cf_ratings(uid, iid, utm, itm, tail_u, tail_i)


# speedup vs baseline: 3.5066x; 1.0867x over previous
import functools

import jax
import jax.numpy as jnp
from jax import lax
from jax.experimental import pallas as pl
from jax.experimental.pallas import tpu as pltpu
from jax.experimental.pallas import tpu_sc as plsc

NC = 2
NS = 16
L = 16
NW = NC * NS

B = 16384
D = 32
V = 1_000_000
VMAIN = 999_936
VTAIL = V - VMAIN
BPW = B // NW
CHUNK = 128
NCH = BPW // CHUNK

TPB = 31
WS = TPB * 128
NTILES = VMAIN // 128
NCOLS = NTILES // TPB
REGION = NTILES * 1024
MSIZE = 4 * REGION
NSLABS = 2 * 4 * NCOLS
SLABS_PER_W = NSLABS // NW
QMAX = NTILES - 1
GSPAN = QMAX * 1024 + 1024

_mesh = plsc.VectorSubcoreMesh(
    core_axis_name="c", subcore_axis_name="s", num_cores=NC, num_subcores=NS
)


@functools.partial(
    pl.kernel,
    out_type=(
        jax.ShapeDtypeStruct((MSIZE // 1024, 8, 128), jnp.float32),
        jax.ShapeDtypeStruct((MSIZE // 1024, 8, 128), jnp.float32),
    ),
    mesh=_mesh,
    compiler_params=pltpu.CompilerParams(
        needs_layout_passes=False, use_tc_tiling_on_sc=True),
    scratch_types=[
        pltpu.VMEM((TPB, 8, 128), jnp.float32),
        pltpu.VMEM((TPB, 8, 128), jnp.float32),
        pltpu.SemaphoreType.DMA,
        pltpu.SemaphoreType.DMA,
        pltpu.SemaphoreType.DMA,
        pltpu.SemaphoreType.DMA,
    ],
)
def _repack(ut_hbm, it_hbm, uout, iout, buf0, buf1, rs0, rs1, ws0, ws1):
    wid = lax.axis_index("s") * NC + lax.axis_index("c")

    def decode(k):
        t = k // (4 * NCOLS)
        rem = k % (4 * NCOLS)
        r = rem // NCOLS
        c = rem % NCOLS
        return t, r * 8, r * NTILES + c * TPB, c * TPB

    def fire_reads(k, buf, rsem):
        t, row8, _, ctile = decode(k)

        @pl.when(t == 0)
        def _():
            for j in range(TPB):
                pltpu.async_copy(
                    ut_hbm.at[pl.ds(row8, 8), pl.ds((ctile + j) * 128, 128)],
                    buf.at[j], rsem)

        @pl.when(t == 1)
        def _():
            for j in range(TPB):
                pltpu.async_copy(
                    it_hbm.at[pl.ds(row8, 8), pl.ds((ctile + j) * 128, 128)],
                    buf.at[j], rsem)

    def drain_reads(k, buf, rsem):
        t, row8, _, ctile = decode(k)
        for j in range(TPB):
            pltpu.make_async_copy(
                ut_hbm.at[pl.ds(row8, 8), pl.ds((ctile + j) * 128, 128)],
                buf.at[j], rsem).wait()

    def fire_write(k, buf, wsem):
        t, _, tile0, _ = decode(k)

        @pl.when(t == 0)
        def _():
            pltpu.async_copy(buf, uout.at[pl.ds(tile0, TPB)], wsem)

        @pl.when(t == 1)
        def _():
            pltpu.async_copy(buf, iout.at[pl.ds(tile0, TPB)], wsem)

    def drain_write(k, buf, wsem):
        _, _, tile0, _ = decode(k)
        pltpu.make_async_copy(buf, uout.at[pl.ds(tile0, TPB)], wsem).wait()

    k0 = wid * SLABS_PER_W
    fire_reads(k0, buf0, rs0)

    NPAIR = (SLABS_PER_W - 1) // 2

    def pair_body(h, carry):
        ke = k0 + 2 * h
        ko = ke + 1
        drain_reads(ke, buf0, rs0)
        fire_write(ke, buf0, ws0)
        lax.cond(h > 0, lambda: drain_write(ko - 2, buf1, ws1), lambda: None)
        fire_reads(ko, buf1, rs1)
        drain_reads(ko, buf1, rs1)
        fire_write(ko, buf1, ws1)

        def prefetch():
            drain_write(ke, buf0, ws0)
            fire_reads(ke + 2, buf0, rs0)
        lax.cond(h < NPAIR - 1, prefetch, lambda: None)
        return carry

    lax.fori_loop(0, NPAIR, pair_body, 0)

    klast = k0 + SLABS_PER_W - 1
    drain_write(klast - 2, buf0, ws0)
    fire_reads(klast, buf0, rs0)
    drain_reads(klast, buf0, rs0)
    fire_write(klast, buf0, ws0)
    drain_write(klast, buf0, ws0)
    drain_write(klast - 1, buf1, ws1)


@functools.partial(
    pl.kernel,
    out_type=jax.ShapeDtypeStruct((B,), jnp.float32),
    mesh=_mesh,
    compiler_params=pltpu.CompilerParams(
        needs_layout_passes=False, use_tc_tiling_on_sc=False),
    scratch_types=[
        pltpu.VMEM((BPW,), jnp.int32),
        pltpu.VMEM((BPW,), jnp.int32),
        pltpu.VMEM((BPW,), jnp.int32),
        pltpu.VMEM((BPW,), jnp.int32),
        pltpu.VMEM((D, BPW), jnp.float32),
        pltpu.VMEM((D, BPW), jnp.float32),
        pltpu.VMEM((VTAIL * D,), jnp.float32),
        pltpu.VMEM((VTAIL * D,), jnp.float32),
        pltpu.VMEM((BPW,), jnp.float32),
        pltpu.SemaphoreType.DMA,
    ],
)
def _cf_ratings(uid_hbm, iid_hbm, ut_hbm, it_hbm, tu_hbm, ti_hbm, out_hbm,
                uidx, iidx, upart, ipart, ucols, icols, utail, itail,
                outv, sem):
    wid = lax.axis_index("s") * NC + lax.axis_index("c")
    base = wid * BPW

    pltpu.sync_copy(uid_hbm.at[pl.ds(base, BPW)], uidx)
    pltpu.sync_copy(iid_hbm.at[pl.ds(base, BPW)], iidx)
    pltpu.sync_copy(tu_hbm, utail)
    pltpu.sync_copy(ti_hbm, itail)

    def part_body(g, carry):
        sl = pl.ds(g * L, L)
        for idv, pv in ((uidx, upart), (iidx, ipart)):
            ids = idv[sl]
            q = jnp.minimum(lax.shift_right_logical(ids, 7), QMAX)
            pv[sl] = q * 1024 + lax.bitwise_and(ids, 127)
        outv[pl.ds(g * L, L)] = jnp.zeros((L,), jnp.float32)
        return carry
    lax.fori_loop(0, BPW // L, part_body, 0)

    def fire(d):
        offs = (d // 8) * REGION + (d % 8) * 128
        for c in range(NCH):
            sl = pl.ds(c * CHUNK, CHUNK)
            pltpu.async_copy(
                ut_hbm.at[pl.ds(offs, GSPAN)].at[upart.at[sl]],
                ucols.at[d, sl], sem)
            pltpu.async_copy(
                it_hbm.at[pl.ds(offs, GSPAN)].at[ipart.at[sl]],
                icols.at[d, sl], sem)

    def drain(d):
        offs = (d // 8) * REGION + (d % 8) * 128
        for c in range(NCH):
            sl = pl.ds(c * CHUNK, CHUNK)
            pltpu.make_async_copy(
                ut_hbm.at[pl.ds(offs, GSPAN)].at[upart.at[sl]],
                ucols.at[d, sl], sem).wait()
            pltpu.make_async_copy(
                it_hbm.at[pl.ds(offs, GSPAN)].at[ipart.at[sl]],
                icols.at[d, sl], sem).wait()

    fire(0)

    def d_loop(d, carry):
        lax.cond(d < D - 1, lambda: fire(d + 1), lambda: None)
        drain(d)

        def group_body(g, c2):
            sl = pl.ds(g * L, L)
            outv[sl] = outv[sl] + ucols[d, sl] * icols[d, sl]
            return c2
        lax.fori_loop(0, BPW // L, group_body, 0)
        return carry

    lax.fori_loop(0, D, d_loop, 0)

    def fix_body(g, carry):
        sl = pl.ds(g * L, L)
        uids = uidx[sl]
        iids = iidx[sl]
        um = uids >= VMAIN
        im = iids >= VMAIN
        any_tail = lax.reduce_max(
            jnp.where(um | im, jnp.int32(1), jnp.int32(0)), axes=(0,))

        @pl.when(any_tail > 0)
        def _():
            ubase = jnp.where(um, uids - VMAIN, 0) * D
            ibase = jnp.where(im, iids - VMAIN, 0) * D
            acc = jnp.zeros((L,), jnp.float32)
            for d in range(D):
                tu = plsc.load_gather(utail, [ubase + d])
                ti = plsc.load_gather(itail, [ibase + d])
                u = jnp.where(um, tu, ucols[d, sl])
                v = jnp.where(im, ti, icols[d, sl])
                acc = acc + u * v
            outv[sl] = acc
        return carry
    lax.fori_loop(0, BPW // L, fix_body, 0)

    pltpu.sync_copy(outv, out_hbm.at[pl.ds(base, BPW)])


def kernel(user_ids, item_ids, user_table, item_table):
    uid = user_ids.astype(jnp.int32)
    iid = item_ids.astype(jnp.int32)
    utm, itm = _repack(user_table.T, item_table.T)
    utm = utm.reshape(MSIZE)
    itm = itm.reshape(MSIZE)
    tail_u = user_table[VMAIN:].reshape(VTAIL * D)
    tail_i = item_table[VMAIN:].reshape(VTAIL * D)
    return _cf_ratings(uid, iid, utm, itm, tail_u, tail_i)


# speedup vs baseline: 3.5822x; 1.0216x over previous
import functools

import jax
import jax.numpy as jnp
from jax import lax
from jax.experimental import pallas as pl
from jax.experimental.pallas import tpu as pltpu
from jax.experimental.pallas import tpu_sc as plsc

NC = 2
NS = 16
L = 16
NW = NC * NS

B = 16384
D = 32
V = 1_000_000
VMAIN = 999_936
VTAIL = V - VMAIN
BPW = B // NW
CHUNK = 128
NCH = BPW // CHUNK

TPB = 31
WS = TPB * 128
NTILES = VMAIN // 128
NCOLS = NTILES // TPB
REGION = NTILES * 1024
MSIZE = 4 * REGION
NSLABS = 2 * 4 * NCOLS
SLABS_PER_W = NSLABS // NW
QMAX = NTILES - 1
GSPAN = QMAX * 1024 + 1024

_mesh = plsc.VectorSubcoreMesh(
    core_axis_name="c", subcore_axis_name="s", num_cores=NC, num_subcores=NS
)


@functools.partial(
    pl.kernel,
    out_type=(
        jax.ShapeDtypeStruct((MSIZE // 1024, 8, 128), jnp.float32),
        jax.ShapeDtypeStruct((MSIZE // 1024, 8, 128), jnp.float32),
    ),
    mesh=_mesh,
    compiler_params=pltpu.CompilerParams(
        needs_layout_passes=False, use_tc_tiling_on_sc=True),
    scratch_types=[
        pltpu.VMEM((TPB, 8, 128), jnp.float32),
        pltpu.VMEM((TPB, 8, 128), jnp.float32),
        pltpu.SemaphoreType.DMA,
        pltpu.SemaphoreType.DMA,
        pltpu.SemaphoreType.DMA,
        pltpu.SemaphoreType.DMA,
    ],
)
def _repack(ut_hbm, it_hbm, uout, iout, buf0, buf1, rs0, rs1, ws0, ws1):
    wid = lax.axis_index("s") * NC + lax.axis_index("c")

    def decode(k):
        t = k // (4 * NCOLS)
        rem = k % (4 * NCOLS)
        r = rem // NCOLS
        c = rem % NCOLS
        return t, r * 8, r * NTILES + c * TPB, c * TPB

    def fire_reads(k, buf, rsem):
        t, row8, _, ctile = decode(k)

        @pl.when(t == 0)
        def _():
            for j in range(TPB):
                pltpu.async_copy(
                    ut_hbm.at[pl.ds(row8, 8), pl.ds((ctile + j) * 128, 128)],
                    buf.at[j], rsem)

        @pl.when(t == 1)
        def _():
            for j in range(TPB):
                pltpu.async_copy(
                    it_hbm.at[pl.ds(row8, 8), pl.ds((ctile + j) * 128, 128)],
                    buf.at[j], rsem)

    def drain_reads(k, buf, rsem):
        t, row8, _, ctile = decode(k)
        for j in range(TPB):
            pltpu.make_async_copy(
                ut_hbm.at[pl.ds(row8, 8), pl.ds((ctile + j) * 128, 128)],
                buf.at[j], rsem).wait()

    def fire_write(k, buf, wsem):
        t, _, tile0, _ = decode(k)

        @pl.when(t == 0)
        def _():
            pltpu.async_copy(buf, uout.at[pl.ds(tile0, TPB)], wsem)

        @pl.when(t == 1)
        def _():
            pltpu.async_copy(buf, iout.at[pl.ds(tile0, TPB)], wsem)

    def drain_write(k, buf, wsem):
        _, _, tile0, _ = decode(k)
        pltpu.make_async_copy(buf, uout.at[pl.ds(tile0, TPB)], wsem).wait()

    k0 = wid * SLABS_PER_W
    fire_reads(k0, buf0, rs0)

    NPAIR = (SLABS_PER_W - 1) // 2

    def pair_body(h, carry):
        ke = k0 + 2 * h
        ko = ke + 1
        drain_reads(ke, buf0, rs0)
        fire_write(ke, buf0, ws0)
        lax.cond(h > 0, lambda: drain_write(ko - 2, buf1, ws1), lambda: None)
        fire_reads(ko, buf1, rs1)
        drain_reads(ko, buf1, rs1)
        fire_write(ko, buf1, ws1)

        def prefetch():
            drain_write(ke, buf0, ws0)
            fire_reads(ke + 2, buf0, rs0)
        lax.cond(h < NPAIR - 1, prefetch, lambda: None)
        return carry

    lax.fori_loop(0, NPAIR, pair_body, 0)

    klast = k0 + SLABS_PER_W - 1
    drain_write(klast - 2, buf0, ws0)
    fire_reads(klast, buf0, rs0)
    drain_reads(klast, buf0, rs0)
    fire_write(klast, buf0, ws0)
    drain_write(klast, buf0, ws0)
    drain_write(klast - 1, buf1, ws1)


@functools.partial(
    pl.kernel,
    out_type=jax.ShapeDtypeStruct((B,), jnp.float32),
    mesh=_mesh,
    compiler_params=pltpu.CompilerParams(
        needs_layout_passes=False, use_tc_tiling_on_sc=False),
    scratch_types=[
        pltpu.VMEM((BPW,), jnp.int32),
        pltpu.VMEM((BPW,), jnp.int32),
        pltpu.VMEM((BPW,), jnp.int32),
        pltpu.VMEM((BPW,), jnp.int32),
        pltpu.VMEM((D, BPW), jnp.float32),
        pltpu.VMEM((D, BPW), jnp.float32),
        pltpu.VMEM((VTAIL * D,), jnp.float32),
        pltpu.VMEM((VTAIL * D,), jnp.float32),
        pltpu.VMEM((BPW,), jnp.float32),
        pltpu.SemaphoreType.DMA,
    ],
)
def _cf_ratings(uid_hbm, iid_hbm, ut_hbm, it_hbm, tu_hbm, ti_hbm, out_hbm,
                uidx, iidx, upart, ipart, ucols, icols, utail, itail,
                outv, sem):
    wid = lax.axis_index("s") * NC + lax.axis_index("c")
    base = wid * BPW

    pltpu.sync_copy(uid_hbm.at[pl.ds(base, BPW)], uidx)
    pltpu.sync_copy(iid_hbm.at[pl.ds(base, BPW)], iidx)
    pltpu.sync_copy(tu_hbm, utail)
    pltpu.sync_copy(ti_hbm, itail)

    def part_body(g, carry):
        sl = pl.ds(g * L, L)
        for idv, pv in ((uidx, upart), (iidx, ipart)):
            ids = idv[sl]
            q = jnp.minimum(lax.shift_right_logical(ids, 7), QMAX)
            pv[sl] = q * 1024 + lax.bitwise_and(ids, 127)
        outv[pl.ds(g * L, L)] = jnp.zeros((L,), jnp.float32)
        return carry
    lax.fori_loop(0, BPW // L, part_body, 0)

    def fire(d):
        offs = (d // 8) * REGION + (d % 8) * 128
        for c in range(NCH):
            sl = pl.ds(c * CHUNK, CHUNK)
            pltpu.async_copy(
                ut_hbm.at[pl.ds(offs, GSPAN)].at[upart.at[sl]],
                ucols.at[d, sl], sem)
            pltpu.async_copy(
                it_hbm.at[pl.ds(offs, GSPAN)].at[ipart.at[sl]],
                icols.at[d, sl], sem)

    def drain(d):
        offs = (d // 8) * REGION + (d % 8) * 128
        for c in range(NCH):
            sl = pl.ds(c * CHUNK, CHUNK)
            pltpu.make_async_copy(
                ut_hbm.at[pl.ds(offs, GSPAN)].at[upart.at[sl]],
                ucols.at[d, sl], sem).wait()
            pltpu.make_async_copy(
                it_hbm.at[pl.ds(offs, GSPAN)].at[ipart.at[sl]],
                icols.at[d, sl], sem).wait()

    fire(0)
    fire(1)

    def d_loop(d, carry):
        lax.cond(d < D - 2, lambda: fire(d + 2), lambda: None)
        drain(d)

        def group_body(g, c2):
            sl = pl.ds(g * L, L)
            outv[sl] = outv[sl] + ucols[d, sl] * icols[d, sl]
            return c2
        lax.fori_loop(0, BPW // L, group_body, 0)
        return carry

    lax.fori_loop(0, D, d_loop, 0)

    def fix_body(g, carry):
        sl = pl.ds(g * L, L)
        uids = uidx[sl]
        iids = iidx[sl]
        um = uids >= VMAIN
        im = iids >= VMAIN
        any_tail = lax.reduce_max(
            jnp.where(um | im, jnp.int32(1), jnp.int32(0)), axes=(0,))

        @pl.when(any_tail > 0)
        def _():
            ubase = jnp.where(um, uids - VMAIN, 0) * D
            ibase = jnp.where(im, iids - VMAIN, 0) * D
            acc = jnp.zeros((L,), jnp.float32)
            for d in range(D):
                tu = plsc.load_gather(utail, [ubase + d])
                ti = plsc.load_gather(itail, [ibase + d])
                u = jnp.where(um, tu, ucols[d, sl])
                v = jnp.where(im, ti, icols[d, sl])
                acc = acc + u * v
            outv[sl] = acc
        return carry
    lax.fori_loop(0, BPW // L, fix_body, 0)

    pltpu.sync_copy(outv, out_hbm.at[pl.ds(base, BPW)])


def kernel(user_ids, item_ids, user_table, item_table):
    uid = user_ids.astype(jnp.int32)
    iid = item_ids.astype(jnp.int32)
    utm, itm = _repack(user_table.T, item_table.T)
    utm = utm.reshape(MSIZE)
    itm = itm.reshape(MSIZE)
    tail_u = user_table[VMAIN:].reshape(VTAIL * D)
    tail_i = item_table[VMAIN:].reshape(VTAIL * D)
    return _cf_ratings(uid, iid, utm, itm, tail_u, tail_i)
